# baseline (speedup 1.0000x reference)
import functools

import jax
import jax.numpy as jnp
from jax import lax
from jax.experimental import pallas as pl
from jax.experimental.pallas import tpu as pltpu, tpu_sc as plsc

N = 10000
E = 320000
D = 128
H = 128
HH = H // 2
C = 32
G = 128

NC = 2
NS = 16
NW = NC * NS
CHUNK = 128
CPT = 176
NQ = 4
CPH = CPT // NQ
EP = NS * CPT * CHUNK
CHUNK_DEG = 128
CPW_DEG = EP // (NW * CHUNK_DEG)
NP = 10240
RPT = NP // NS

_mesh = plsc.VectorSubcoreMesh(core_axis_name="c", subcore_axis_name="s")


@functools.partial(
    pl.kernel,
    out_type=jax.ShapeDtypeStruct((NC, NP), jnp.float32),
    mesh=_mesh,
    scratch_types=[
        pltpu.VMEM((CPW_DEG, CHUNK_DEG), jnp.int32),
        pltpu.VMEM((CHUNK_DEG,), jnp.float32),
        pltpu.VMEM_SHARED((NP,), jnp.float32),
    ],
)
def _sc_degree(dst_hbm, zeros1_hbm, deg_out, dst_v, ones_v, acc):
    cid = lax.axis_index("c")
    sid = lax.axis_index("s")
    wid = sid * NC + cid
    pltpu.sync_copy(zeros1_hbm, acc.at[pl.ds(sid * RPT, RPT)])
    pltpu.sync_copy(dst_hbm.at[wid], dst_v)
    for i in range(CHUNK_DEG // 16):
        ones_v[pl.ds(i * 16, 16)] = jnp.full((16,), 1.0, dtype=jnp.float32)
    plsc.subcore_barrier()

    def body(j, carry):
        pltpu.sync_copy(ones_v, acc.at[dst_v.at[j]], add=True)
        return carry

    lax.fori_loop(0, CPW_DEG, body, 0)
    plsc.subcore_barrier()
    pltpu.sync_copy(acc.at[pl.ds(sid * RPT, RPT)],
                    deg_out.at[cid, pl.ds(sid * RPT, RPT)])


@functools.partial(
    pl.kernel,
    out_type=jax.ShapeDtypeStruct((NC, NP, HH), jnp.float32),
    mesh=_mesh,
    scratch_types=[
        pltpu.VMEM((CPH, CHUNK), jnp.int32),
        pltpu.VMEM((CPH, CHUNK), jnp.int32),
        pltpu.VMEM((CHUNK, HH), jnp.float32),
        pltpu.VMEM((CHUNK, HH), jnp.float32),
        pltpu.VMEM((CHUNK, HH), jnp.float32),
        pltpu.VMEM((CHUNK, HH), jnp.float32),
        pltpu.VMEM_SHARED((NP, HH), jnp.float32),
        pltpu.VMEM_SHARED((NP, HH), jnp.float32),
        pltpu.SemaphoreType.DMA((4,)),
        pltpu.SemaphoreType.DMA((4,)),
    ],
    compiler_params=pltpu.CompilerParams(use_tc_tiling_on_sc=False),
)
def _sc_spmm(src_hbm, dst_hbm, h_hbm, zeros2_hbm, out_hbm,
             src_v, dst_v, r0, r1, r2, r3, table, acc, sem_g, sem_s):
    rows = (r0, r1, r2, r3)
    cid = lax.axis_index("c")
    sid = lax.axis_index("s")
    rs = pl.ds(sid * RPT, RPT)
    pltpu.sync_copy(h_hbm.at[cid, rs], table.at[rs])
    pltpu.sync_copy(zeros2_hbm, acc.at[rs])
    plsc.subcore_barrier()

    def run_part():
        for b in range(2):
            pltpu.async_copy(table.at[src_v.at[b]], rows[b], sem_g.at[b])

        def body(i, carry):
            for b in range(4):
                j = 4 * i + b
                sl = b
                sn = (b + 2) % 4

                @pl.when(j >= 2)
                def _():
                    pltpu.make_async_copy(
                        rows[sn], acc.at[dst_v.at[j - 2]],
                        sem_s.at[sn]).wait()

                @pl.when(j + 2 < CPH)
                def _():
                    pltpu.async_copy(
                        table.at[src_v.at[j + 2]], rows[sn], sem_g.at[sn])

                pltpu.make_async_copy(
                    table.at[src_v.at[j]], rows[sl], sem_g.at[sl]).wait()
                pltpu.async_copy(
                    rows[sl], acc.at[dst_v.at[j]], sem_s.at[sl], add=True)
            return carry

        lax.fori_loop(0, CPH // 4, body, 0)
        for j in (CPH - 2, CPH - 1):
            pltpu.make_async_copy(
                rows[j % 4], acc.at[dst_v.at[j]], sem_s.at[j % 4]).wait()

    for part in range(NQ):
        pltpu.sync_copy(src_hbm.at[sid, pl.ds(part * CPH, CPH)], src_v)
        pltpu.sync_copy(dst_hbm.at[sid, pl.ds(part * CPH, CPH)], dst_v)
        run_part()
    plsc.subcore_barrier()
    pltpu.sync_copy(acc.at[rs], out_hbm.at[cid, rs])


def _split_halves(h):
    return jnp.stack([h[:, :HH], h[:, HH:]])


def _tc1_body(degp_ref, xp_ref, w1_ref, dis_ref, h1_ref):
    deg = (degp_ref[0] + degp_ref[1]) + 1.0
    dis = lax.rsqrt(deg)
    dis_ref[...] = dis
    h = jnp.dot(xp_ref[...], w1_ref[...], preferred_element_type=jnp.float32)
    h1_ref[...] = _split_halves(h * dis[:, None])


def _tc2_body(acc_ref, h1p_ref, dis_ref, b1_ref, w2_ref, h2p_ref):
    agg = acc_ref[...] + h1p_ref[...]
    full = jnp.concatenate([agg[0], agg[1]], axis=-1)
    dis = dis_ref[...]
    h1 = jnp.maximum(full * dis[:, None] + b1_ref[...][None, :], 0.0)
    h = jnp.dot(h1, w2_ref[...], preferred_element_type=jnp.float32)
    h2p_ref[...] = _split_halves(h * dis[:, None])


def _tc3_body(acc_ref, h2p_ref, dis_ref, b2_ref, batchp_ref, wfc_ref,
              bfc_ref, out_ref):
    agg = acc_ref[...] + h2p_ref[...]
    full = jnp.concatenate([agg[0], agg[1]], axis=-1)
    dis = dis_ref[...]
    h2 = jnp.maximum(full * dis[:, None] + b2_ref[...][None, :], 0.0)
    gid = lax.broadcasted_iota(jnp.int32, (G, NP), 0)
    p = (batchp_ref[...][None, :] == gid).astype(jnp.float32)
    sums = jnp.dot(p, h2, preferred_element_type=jnp.float32)
    counts = jnp.sum(p, axis=1)
    pooled = sums / jnp.maximum(counts, 1.0)[:, None]
    out_ref[...] = (jnp.dot(pooled, wfc_ref[...],
                            preferred_element_type=jnp.float32)
                    + bfc_ref[...][None, :])


def kernel(x, edge_index, batch, W1, b1, W2, b2, Wfc, bfc):
    src = edge_index[0]
    dst = edge_index[1]
    pad = EP - E
    srcp = jnp.concatenate([src, jnp.zeros((pad,), jnp.int32)])
    dstp = jnp.concatenate([dst, jnp.full((pad,), N, jnp.int32)])
    src16 = srcp.reshape(NS, CPT, CHUNK)
    dst16 = dstp.reshape(NS, CPT, CHUNK)
    dst32 = dstp.reshape(NW, CPW_DEG, CHUNK_DEG)
    xp = jnp.pad(x, ((0, NP - N), (0, 0)))
    batchp = jnp.pad(batch, (0, NP - N), constant_values=G)
    zeros1 = jnp.zeros((RPT,), jnp.float32)
    zeros2 = jnp.zeros((RPT, HH), jnp.float32)

    degp = _sc_degree(dst32, zeros1)

    dis, h1p = pl.pallas_call(
        _tc1_body,
        out_shape=(jax.ShapeDtypeStruct((NP,), jnp.float32),
                   jax.ShapeDtypeStruct((NC, NP, HH), jnp.float32)),
    )(degp, xp, W1)

    acc1 = _sc_spmm(src16, dst16, h1p, zeros2)

    h2p = pl.pallas_call(
        _tc2_body,
        out_shape=jax.ShapeDtypeStruct((NC, NP, HH), jnp.float32),
    )(acc1, h1p, dis, b1, W2)

    acc2 = _sc_spmm(src16, dst16, h2p, zeros2)

    out = pl.pallas_call(
        _tc3_body,
        out_shape=jax.ShapeDtypeStruct((G, C), jnp.float32),
    )(acc2, h2p, dis, b2, batchp, Wfc, bfc)

    return out


# speedup vs baseline: 1.2094x; 1.2094x over previous
import functools

import jax
import jax.numpy as jnp
from jax import lax
from jax.experimental import pallas as pl
from jax.experimental.pallas import tpu as pltpu, tpu_sc as plsc

N = 10000
E = 320000
D = 128
H = 128
HH = H // 2
C = 32
G = 128

NC = 2
NS = 16
NW = NC * NS
CHUNK = 128
CPT = 159
CPH = 81
PARTS = (81, 78)
EP = NS * CPT * CHUNK
CHUNK_DEG = 128
CPW_DEG = 80
EP_DEG = NW * CPW_DEG * CHUNK_DEG
NP = 10240
RPT = NP // NS

_mesh = plsc.VectorSubcoreMesh(core_axis_name="c", subcore_axis_name="s")


@functools.partial(
    pl.kernel,
    out_type=jax.ShapeDtypeStruct((NC, NP), jnp.float32),
    mesh=_mesh,
    scratch_types=[
        pltpu.VMEM((CPW_DEG, CHUNK_DEG), jnp.int32),
        pltpu.VMEM((CHUNK_DEG,), jnp.float32),
        pltpu.VMEM_SHARED((NP,), jnp.float32),
    ],
)
def _sc_degree(dst_hbm, zeros1_hbm, deg_out, dst_v, ones_v, acc):
    cid = lax.axis_index("c")
    sid = lax.axis_index("s")
    wid = sid * NC + cid
    pltpu.sync_copy(zeros1_hbm, acc.at[pl.ds(sid * RPT, RPT)])
    pltpu.sync_copy(dst_hbm.at[wid], dst_v)
    for i in range(CHUNK_DEG // 16):
        ones_v[pl.ds(i * 16, 16)] = jnp.full((16,), 1.0, dtype=jnp.float32)
    plsc.subcore_barrier()

    def body(j, carry):
        pltpu.sync_copy(ones_v, acc.at[dst_v.at[j]], add=True)
        return carry

    lax.fori_loop(0, CPW_DEG, body, 0)
    plsc.subcore_barrier()
    pltpu.sync_copy(acc.at[pl.ds(sid * RPT, RPT)],
                    deg_out.at[cid, pl.ds(sid * RPT, RPT)])


@functools.partial(
    pl.kernel,
    out_type=jax.ShapeDtypeStruct((NC, NP, HH), jnp.float32),
    mesh=_mesh,
    scratch_types=[
        pltpu.VMEM((CPH, CHUNK), jnp.int32),
        pltpu.VMEM((CPH, CHUNK), jnp.int32),
        pltpu.VMEM((CHUNK, HH), jnp.float32),
        pltpu.VMEM((CHUNK, HH), jnp.float32),
        pltpu.VMEM((CHUNK, HH), jnp.float32),
        pltpu.VMEM_SHARED((NP, HH), jnp.float32),
        pltpu.VMEM_SHARED((NP, HH), jnp.float32),
        pltpu.SemaphoreType.DMA((3,)),
        pltpu.SemaphoreType.DMA((3,)),
    ],
    compiler_params=pltpu.CompilerParams(use_tc_tiling_on_sc=False),
)
def _sc_spmm(src_hbm, dst_hbm, h_hbm, zeros2_hbm, out_hbm,
             src_v, dst_v, r0, r1, r2, table, acc, sem_g, sem_s):
    rows = (r0, r1, r2)
    cid = lax.axis_index("c")
    sid = lax.axis_index("s")
    rs = pl.ds(sid * RPT, RPT)
    pltpu.sync_copy(h_hbm.at[cid, rs], table.at[rs])
    pltpu.sync_copy(zeros2_hbm, acc.at[rs])
    plsc.subcore_barrier()

    def gwait(j, sl):
        pltpu.make_async_copy(table.at[src_v.at[j]], rows[sl],
                              sem_g.at[sl]).wait()

    def gfire(j, sl):
        pltpu.async_copy(table.at[src_v.at[j]], rows[sl], sem_g.at[sl])

    def swait(j, sl):
        pltpu.make_async_copy(rows[sl], acc.at[dst_v.at[j]],
                              sem_s.at[sl]).wait()

    def sfire(j, sl):
        pltpu.async_copy(rows[sl], acc.at[dst_v.at[j]], sem_s.at[sl],
                         add=True)

    def run_part(n):
        gfire(0, 0)
        gfire(1, 1)
        gwait(0, 0)
        sfire(0, 0)
        gfire(2, 2)
        gwait(1, 1)
        sfire(1, 1)

        def body(i, carry):
            for b in range(3):
                j = 3 * i + 2 + b
                sl = (2 + b) % 3
                sn = (sl + 1) % 3
                swait(j - 2, sn)
                gfire(j + 1, sn)
                gwait(j, sl)
                sfire(j, sl)
            return carry

        lax.fori_loop(0, (n - 3) // 3, body, 0)
        swait(n - 3, 0)
        gwait(n - 1, 2)
        sfire(n - 1, 2)
        swait(n - 2, 1)
        swait(n - 1, 2)

    base = 0
    for n in PARTS:
        pltpu.sync_copy(src_hbm.at[sid, pl.ds(base, n)],
                        src_v.at[pl.ds(0, n)])
        pltpu.sync_copy(dst_hbm.at[sid, pl.ds(base, n)],
                        dst_v.at[pl.ds(0, n)])
        run_part(n)
        base += n
    plsc.subcore_barrier()
    pltpu.sync_copy(acc.at[rs], out_hbm.at[cid, rs])


def _split_halves(h):
    return jnp.stack([h[:, :HH], h[:, HH:]])


def _tc1_body(degp_ref, xp_ref, w1_ref, dis_ref, h1_ref):
    deg = (degp_ref[0] + degp_ref[1]) + 1.0
    dis = lax.rsqrt(deg)
    dis_ref[...] = dis
    h = jnp.dot(xp_ref[...], w1_ref[...], preferred_element_type=jnp.float32)
    h1_ref[...] = _split_halves(h * dis[:, None])


def _tc2_body(acc_ref, h1p_ref, dis_ref, b1_ref, w2_ref, h2p_ref):
    agg = acc_ref[...] + h1p_ref[...]
    full = jnp.concatenate([agg[0], agg[1]], axis=-1)
    dis = dis_ref[...]
    h1 = jnp.maximum(full * dis[:, None] + b1_ref[...][None, :], 0.0)
    h = jnp.dot(h1, w2_ref[...], preferred_element_type=jnp.float32)
    h2p_ref[...] = _split_halves(h * dis[:, None])


def _tc3_body(acc_ref, h2p_ref, dis_ref, b2_ref, batchp_ref, wfc_ref,
              bfc_ref, out_ref):
    agg = acc_ref[...] + h2p_ref[...]
    full = jnp.concatenate([agg[0], agg[1]], axis=-1)
    dis = dis_ref[...]
    h2 = jnp.maximum(full * dis[:, None] + b2_ref[...][None, :], 0.0)
    gid = lax.broadcasted_iota(jnp.int32, (G, NP), 0)
    p = (batchp_ref[...][None, :] == gid).astype(jnp.float32)
    sums = jnp.dot(p, h2, preferred_element_type=jnp.float32)
    counts = jnp.sum(p, axis=1)
    pooled = sums / jnp.maximum(counts, 1.0)[:, None]
    out_ref[...] = (jnp.dot(pooled, wfc_ref[...],
                            preferred_element_type=jnp.float32)
                    + bfc_ref[...][None, :])


def kernel(x, edge_index, batch, W1, b1, W2, b2, Wfc, bfc):
    src = edge_index[0]
    dst = edge_index[1]
    srcp = jnp.concatenate([src, jnp.zeros((EP - E,), jnp.int32)])
    dstp = jnp.concatenate([dst, jnp.full((EP - E,), N, jnp.int32)])
    src16 = srcp.reshape(NS, CPT, CHUNK)
    dst16 = dstp.reshape(NS, CPT, CHUNK)
    dst32 = jnp.concatenate(
        [dst, jnp.full((EP_DEG - E,), N, jnp.int32)]).reshape(
            NW, CPW_DEG, CHUNK_DEG)
    xp = jnp.pad(x, ((0, NP - N), (0, 0)))
    batchp = jnp.pad(batch, (0, NP - N), constant_values=G)
    zeros1 = jnp.zeros((RPT,), jnp.float32)
    zeros2 = jnp.zeros((RPT, HH), jnp.float32)

    degp = _sc_degree(dst32, zeros1)

    dis, h1p = pl.pallas_call(
        _tc1_body,
        out_shape=(jax.ShapeDtypeStruct((NP,), jnp.float32),
                   jax.ShapeDtypeStruct((NC, NP, HH), jnp.float32)),
    )(degp, xp, W1)

    acc1 = _sc_spmm(src16, dst16, h1p, zeros2)

    h2p = pl.pallas_call(
        _tc2_body,
        out_shape=jax.ShapeDtypeStruct((NC, NP, HH), jnp.float32),
    )(acc1, h1p, dis, b1, W2)

    acc2 = _sc_spmm(src16, dst16, h2p, zeros2)

    out = pl.pallas_call(
        _tc3_body,
        out_shape=jax.ShapeDtypeStruct((G, C), jnp.float32),
    )(acc2, h2p, dis, b2, batchp, Wfc, bfc)

    return out


# speedup vs baseline: 1.2126x; 1.0027x over previous
import functools

import jax
import jax.numpy as jnp
from jax import lax
from jax.experimental import pallas as pl
from jax.experimental.pallas import tpu as pltpu, tpu_sc as plsc

N = 10000
E = 320000
D = 128
H = 128
HH = H // 2
C = 32
G = 128

NC = 2
NS = 16
NW = NC * NS
CHUNK = 128
CPT = 159
CPH = 81
PARTS = (81, 78)
EP = NS * CPT * CHUNK
CHUNK_DEG = 128
CPW_DEG = 80
EP_DEG = NW * CPW_DEG * CHUNK_DEG
NP = 10240
RPT = NP // NS

_mesh = plsc.VectorSubcoreMesh(core_axis_name="c", subcore_axis_name="s")


@functools.partial(
    pl.kernel,
    out_type=jax.ShapeDtypeStruct((NC, NP), jnp.float32),
    mesh=_mesh,
    scratch_types=[
        pltpu.VMEM((CPW_DEG, CHUNK_DEG), jnp.int32),
        pltpu.VMEM((CHUNK_DEG,), jnp.float32),
        pltpu.VMEM_SHARED((NP,), jnp.float32),
    ],
)
def _sc_degree(dst_hbm, zeros1_hbm, deg_out, dst_v, ones_v, acc):
    cid = lax.axis_index("c")
    sid = lax.axis_index("s")
    wid = sid * NC + cid
    pltpu.sync_copy(zeros1_hbm, acc.at[pl.ds(sid * RPT, RPT)])
    pltpu.sync_copy(dst_hbm.at[wid], dst_v)
    for i in range(CHUNK_DEG // 16):
        ones_v[pl.ds(i * 16, 16)] = jnp.full((16,), 1.0, dtype=jnp.float32)
    plsc.subcore_barrier()

    def body(j, carry):
        pltpu.sync_copy(ones_v, acc.at[dst_v.at[j]], add=True)
        return carry

    lax.fori_loop(0, CPW_DEG, body, 0)
    plsc.subcore_barrier()
    pltpu.sync_copy(acc.at[pl.ds(sid * RPT, RPT)],
                    deg_out.at[cid, pl.ds(sid * RPT, RPT)])


@functools.partial(
    pl.kernel,
    out_type=jax.ShapeDtypeStruct((NC, NP, HH), jnp.float32),
    mesh=_mesh,
    scratch_types=[
        pltpu.VMEM((CPH, CHUNK), jnp.int32),
        pltpu.VMEM((CPH, CHUNK), jnp.int32),
        pltpu.VMEM((CHUNK, HH), jnp.float32),
        pltpu.VMEM((CHUNK, HH), jnp.float32),
        pltpu.VMEM((CHUNK, HH), jnp.float32),
        pltpu.VMEM_SHARED((NP, HH), jnp.float32),
        pltpu.VMEM_SHARED((NP, HH), jnp.float32),
        pltpu.SemaphoreType.DMA((3,)),
        pltpu.SemaphoreType.DMA((3,)),
    ],
    compiler_params=pltpu.CompilerParams(use_tc_tiling_on_sc=False),
)
def _sc_spmm(src_hbm, dst_hbm, h_hbm, zeros2_hbm, out_hbm,
             src_v, dst_v, r0, r1, r2, table, acc, sem_g, sem_s):
    rows = (r0, r1, r2)
    cid = lax.axis_index("c")
    sid = lax.axis_index("s")
    rs = pl.ds(sid * RPT, RPT)
    pltpu.sync_copy(h_hbm.at[cid, rs], table.at[rs])
    pltpu.sync_copy(zeros2_hbm, acc.at[rs])
    plsc.subcore_barrier()

    def gwait(j, sl):
        pltpu.make_async_copy(table.at[src_v.at[j]], rows[sl],
                              sem_g.at[sl]).wait()

    def gfire(j, sl):
        pltpu.async_copy(table.at[src_v.at[j]], rows[sl], sem_g.at[sl])

    def swait(j, sl):
        pltpu.make_async_copy(rows[sl], acc.at[dst_v.at[j]],
                              sem_s.at[sl]).wait()

    def sfire(j, sl):
        pltpu.async_copy(rows[sl], acc.at[dst_v.at[j]], sem_s.at[sl],
                         add=True)

    def run_part(n):
        gfire(0, 0)
        gfire(1, 1)
        gwait(0, 0)
        sfire(0, 0)
        gfire(2, 2)
        gwait(1, 1)
        sfire(1, 1)

        def body(i, carry):
            for b in range(3):
                j = 3 * i + 2 + b
                sl = (2 + b) % 3
                sn = (sl + 1) % 3
                swait(j - 2, sn)
                gfire(j + 1, sn)
                gwait(j, sl)
                sfire(j, sl)
            return carry

        lax.fori_loop(0, (n - 3) // 3, body, 0)
        swait(n - 3, 0)
        gwait(n - 1, 2)
        sfire(n - 1, 2)
        swait(n - 2, 1)
        swait(n - 1, 2)

    base = 0
    for n in PARTS:
        pltpu.sync_copy(src_hbm.at[sid, pl.ds(base, n)],
                        src_v.at[pl.ds(0, n)])
        pltpu.sync_copy(dst_hbm.at[sid, pl.ds(base, n)],
                        dst_v.at[pl.ds(0, n)])
        run_part(n)
        base += n
    plsc.subcore_barrier()
    pltpu.sync_copy(acc.at[rs], out_hbm.at[cid, rs])


def _split_halves(h):
    return jnp.stack([h[:, :HH], h[:, HH:]])


def _tc0_body(xp_ref, w1_ref, g1_ref):
    g1_ref[...] = jnp.dot(xp_ref[...], w1_ref[...],
                          preferred_element_type=jnp.float32)


def _tc1_body(degp_ref, g1_ref, dis_ref, h1_ref):
    deg = (degp_ref[0] + degp_ref[1]) + 1.0
    dis = lax.rsqrt(deg)
    dis_ref[...] = dis
    h1_ref[...] = _split_halves(g1_ref[...] * dis[:, None])


def _tc2_body(acc_ref, h1p_ref, dis_ref, b1_ref, w2_ref, h2p_ref):
    agg = acc_ref[...] + h1p_ref[...]
    full = jnp.concatenate([agg[0], agg[1]], axis=-1)
    dis = dis_ref[...]
    h1 = jnp.maximum(full * dis[:, None] + b1_ref[...][None, :], 0.0)
    h = jnp.dot(h1, w2_ref[...], preferred_element_type=jnp.float32)
    h2p_ref[...] = _split_halves(h * dis[:, None])


def _tc3_body(acc_ref, h2p_ref, dis_ref, b2_ref, batchp_ref, wfc_ref,
              bfc_ref, out_ref):
    agg = acc_ref[...] + h2p_ref[...]
    full = jnp.concatenate([agg[0], agg[1]], axis=-1)
    dis = dis_ref[...]
    h2 = jnp.maximum(full * dis[:, None] + b2_ref[...][None, :], 0.0)
    gid = lax.broadcasted_iota(jnp.int32, (G, NP), 0)
    p = (batchp_ref[...][None, :] == gid).astype(jnp.float32)
    sums = jnp.dot(p, h2, preferred_element_type=jnp.float32)
    counts = jnp.sum(p, axis=1)
    pooled = sums / jnp.maximum(counts, 1.0)[:, None]
    out_ref[...] = (jnp.dot(pooled, wfc_ref[...],
                            preferred_element_type=jnp.float32)
                    + bfc_ref[...][None, :])


def kernel(x, edge_index, batch, W1, b1, W2, b2, Wfc, bfc):
    src = edge_index[0]
    dst = edge_index[1]
    srcp = jnp.concatenate([src, jnp.zeros((EP - E,), jnp.int32)])
    dstp = jnp.concatenate([dst, jnp.full((EP - E,), N, jnp.int32)])
    src16 = srcp.reshape(NS, CPT, CHUNK)
    dst16 = dstp.reshape(NS, CPT, CHUNK)
    dst32 = jnp.concatenate(
        [dst, jnp.full((EP_DEG - E,), N, jnp.int32)]).reshape(
            NW, CPW_DEG, CHUNK_DEG)
    xp = jnp.pad(x, ((0, NP - N), (0, 0)))
    batchp = jnp.pad(batch, (0, NP - N), constant_values=G)
    zeros1 = jnp.zeros((RPT,), jnp.float32)
    zeros2 = jnp.zeros((RPT, HH), jnp.float32)

    g1 = pl.pallas_call(
        _tc0_body,
        out_shape=jax.ShapeDtypeStruct((NP, H), jnp.float32),
    )(xp, W1)
    degp = _sc_degree(dst32, zeros1)

    dis, h1p = pl.pallas_call(
        _tc1_body,
        out_shape=(jax.ShapeDtypeStruct((NP,), jnp.float32),
                   jax.ShapeDtypeStruct((NC, NP, HH), jnp.float32)),
    )(degp, g1)

    acc1 = _sc_spmm(src16, dst16, h1p, zeros2)

    h2p = pl.pallas_call(
        _tc2_body,
        out_shape=jax.ShapeDtypeStruct((NC, NP, HH), jnp.float32),
    )(acc1, h1p, dis, b1, W2)

    acc2 = _sc_spmm(src16, dst16, h2p, zeros2)

    out = pl.pallas_call(
        _tc3_body,
        out_shape=jax.ShapeDtypeStruct((G, C), jnp.float32),
    )(acc2, h2p, dis, b2, batchp, Wfc, bfc)

    return out
